# baseline (speedup 1.0000x reference)
import functools

import jax
import jax.numpy as jnp
from jax import lax
from jax.experimental import pallas as pl
from jax.experimental.pallas import tpu as pltpu
from jax.experimental.pallas import tpu_sc as plsc

_B, _C, _N, _K = 8, 128, 2048, 20
_BN = 256
_NW = 32
_WPB = _NW // _B
_CPW = _C // _WPB
_NOCT = _CPW // 8


def _topk_body(xb_ref, xr_ref, idx_ref):
    xb = xb_ref[0]
    xr = xr_ref[0]
    inner2 = 2.0 * lax.dot_general(
        xr, xb, (((0,), (0,)), ((), ())),
        preferred_element_type=jnp.float32)
    xx = jnp.sum(xb * xb, axis=0, keepdims=True)
    d = inner2 - xx
    iota_f = lax.broadcasted_iota(jnp.int32, (_BN, _N), 1).astype(
        jnp.float32)
    kiota = lax.broadcasted_iota(jnp.int32, (_BN, _K), 1)
    acc = jnp.zeros((_BN, _K), jnp.int32)
    neg_inf = jnp.float32(-jnp.inf)
    big = jnp.float32(_N)
    S, W = 16, _N // 16

    def _tree(op, parts):
        while len(parts) > 1:
            parts = [op(parts[i], parts[i + 1]) if i + 1 < len(parts)
                     else parts[i] for i in range(0, len(parts), 2)]
        return parts[0]

    for t in range(_K):
        m1 = _tree(jnp.maximum, [d[:, i * W:(i + 1) * W] for i in range(S)])
        m = jnp.max(m1, axis=1, keepdims=True)
        cand = jnp.where(d == m, iota_f, big)
        c1 = _tree(jnp.minimum,
                   [cand[:, i * W:(i + 1) * W] for i in range(S)])
        am = jnp.min(c1, axis=1, keepdims=True)
        acc = jnp.where(kiota == t, am.astype(jnp.int32), acc)
        d = jnp.where(cand == am, neg_inf, d)
    idx_ref[0] = acc


_topk = pl.pallas_call(
    _topk_body,
    grid=(_B, _N // _BN),
    in_specs=[
        pl.BlockSpec((1, _C, _N), lambda b, r: (b, 0, 0)),
        pl.BlockSpec((1, _C, _BN), lambda b, r: (b, 0, r)),
    ],
    out_specs=pl.BlockSpec((1, _BN, _K), lambda b, r: (b, r, 0)),
    out_shape=jax.ShapeDtypeStruct((_B, _N, _K), jnp.int32),
)


def _make_sc_gather():
    mesh = plsc.VectorSubcoreMesh(core_axis_name="c", subcore_axis_name="s")

    @functools.partial(
        pl.kernel,
        mesh=mesh,
        compiler_params=pltpu.CompilerParams(needs_layout_passes=False),
        out_type=jax.ShapeDtypeStruct((_B, _K, 2 * _C, _N), jnp.float32),
        scratch_types=[
            pltpu.VMEM((_K, _N), jnp.int32),
            pltpu.VMEM((8, _N), jnp.float32),
            pltpu.VMEM((2, 8, _N), jnp.float32),
            pltpu.SemaphoreType.DMA,
            pltpu.SemaphoreType.DMA,
            pltpu.SemaphoreType.DMA,
        ],
    )
    def sc_gather(x_hbm, idxt_hbm, out_hbm, idxt_v, x8_v, dbuf_v,
                  sem_d0, sem_d1, sem_c):
        cid = lax.axis_index("c")
        sid = lax.axis_index("s")
        wid = sid * 2 + cid
        b = wid // _WPB
        cgrp = wid % _WPB
        pltpu.sync_copy(idxt_hbm.at[b], idxt_v)
        splats = [jnp.full((16,), r, jnp.int32) for r in range(8)]

        def oct_body(co, carry):
            c0 = cgrp * _CPW + co * 8
            pltpu.sync_copy(x_hbm.at[b, pl.ds(c0, 8)], x8_v)
            pend = []
            for j in range(_K):
                p = j % 2
                sem_d = sem_d0 if p == 0 else sem_d1
                pend.append(pltpu.async_copy(
                    x8_v, out_hbm.at[b, j, pl.ds(c0, 8)], sem_c))
                if j >= 2:
                    pend[_idx_d[j - 2]].wait()

                def vbody(vi, carry2):
                    base = pl.multiple_of(vi * 64, 64)
                    for q in range(4):
                        off = base + q * 16
                        vidx = idxt_v[j, pl.ds(off, 16)]
                        for r in range(8):
                            nb = plsc.load_gather(x8_v, [splats[r], vidx])
                            cv = x8_v[r, pl.ds(off, 16)]
                            dbuf_v[p, r, pl.ds(off, 16)] = cv - nb
                    return carry2
                lax.fori_loop(0, _N // 64, vbody, 0)
                _idx_d[j] = len(pend)
                pend.append(pltpu.async_copy(
                    dbuf_v.at[p], out_hbm.at[b, j, pl.ds(_C + c0, 8)], sem_d))
            pend[_idx_d[_K - 2]].wait()
            pend[_idx_d[_K - 1]].wait()
            for j in range(_K):
                pend[2 * j].wait()
            return carry

        _idx_d = {}
        lax.fori_loop(0, _NOCT, oct_body, 0)

    return sc_gather


_sc_gather_cache = []


def kernel(x, k):
    del k
    if not _sc_gather_cache:
        _sc_gather_cache.append(_make_sc_gather())
    idx = _topk(x, x)
    idxt = jnp.transpose(idx, (0, 2, 1))
    phys = _sc_gather_cache[0](x, idxt)
    return jnp.transpose(phys, (0, 2, 3, 1))


# speedup vs baseline: 1.5279x; 1.5279x over previous
import functools

import jax
import jax.numpy as jnp
from jax import lax
from jax.experimental import pallas as pl
from jax.experimental.pallas import tpu as pltpu
from jax.experimental.pallas import tpu_sc as plsc

_B, _C, _N, _K = 8, 128, 2048, 20
_BN = 256
_NW = 32
_WPB = _NW // _B
_CPW = _C // _WPB
_NOCT = _CPW // 8


def _topk_body(xb_ref, xr_ref, idx_ref):
    xb = xb_ref[0]
    xr = xr_ref[0]
    inner2 = 2.0 * lax.dot_general(
        xr, xb, (((0,), (0,)), ((), ())),
        preferred_element_type=jnp.float32)
    xx = jnp.sum(xb * xb, axis=0, keepdims=True)
    d = inner2 - xx
    iota_f = lax.broadcasted_iota(jnp.int32, (_BN, _N), 1).astype(
        jnp.float32)
    kiota = lax.broadcasted_iota(jnp.int32, (_BN, _K), 1)
    acc = jnp.zeros((_BN, _K), jnp.int32)
    neg_inf = jnp.float32(-jnp.inf)
    big = jnp.float32(_N)
    S, W = 16, _N // 16

    def _tree(op, parts):
        while len(parts) > 1:
            parts = [op(parts[i], parts[i + 1]) if i + 1 < len(parts)
                     else parts[i] for i in range(0, len(parts), 2)]
        return parts[0]

    for t in range(_K):
        m1 = _tree(jnp.maximum, [d[:, i * W:(i + 1) * W] for i in range(S)])
        m = jnp.max(m1, axis=1, keepdims=True)
        cand = jnp.where(d == m, iota_f, big)
        c1 = _tree(jnp.minimum,
                   [cand[:, i * W:(i + 1) * W] for i in range(S)])
        am = jnp.min(c1, axis=1, keepdims=True)
        acc = jnp.where(kiota == t, am.astype(jnp.int32), acc)
        d = jnp.where(cand == am, neg_inf, d)
    idx_ref[0] = acc


_topk = pl.pallas_call(
    _topk_body,
    grid=(_B, _N // _BN),
    in_specs=[
        pl.BlockSpec((1, _C, _N), lambda b, r: (b, 0, 0)),
        pl.BlockSpec((1, _C, _BN), lambda b, r: (b, 0, r)),
    ],
    out_specs=pl.BlockSpec((1, _BN, _K), lambda b, r: (b, r, 0)),
    out_shape=jax.ShapeDtypeStruct((_B, _N, _K), jnp.int32),
)


def _make_sc_gather():
    mesh = plsc.VectorSubcoreMesh(core_axis_name="c", subcore_axis_name="s")

    @functools.partial(
        pl.kernel,
        mesh=mesh,
        compiler_params=pltpu.CompilerParams(needs_layout_passes=False),
        out_type=jax.ShapeDtypeStruct((_B, _K, 2 * _C, _N), jnp.float32),
        scratch_types=[
            pltpu.VMEM((_K, _N), jnp.int32),
            pltpu.VMEM((8, _N), jnp.float32),
            pltpu.VMEM((2, 8, _N), jnp.float32),
            pltpu.SemaphoreType.DMA,
            pltpu.SemaphoreType.DMA,
            pltpu.SemaphoreType.DMA,
        ],
    )
    def sc_gather(x_hbm, idxt_hbm, out_hbm, idxt_v, x8_v, dbuf_v,
                  sem_d0, sem_d1, sem_c):
        cid = lax.axis_index("c")
        sid = lax.axis_index("s")
        wid = sid * 2 + cid
        b = wid // _WPB
        cgrp = wid % _WPB
        pltpu.sync_copy(idxt_hbm.at[b], idxt_v)
        splats = [jnp.full((16,), r, jnp.int32) for r in range(8)]

        def oct_body(co, carry):
            c0 = cgrp * _CPW + co * 8
            pltpu.sync_copy(x_hbm.at[b, pl.ds(c0, 8)], x8_v)
            pend = []
            for j in range(_K):
                p = j % 2
                sem_d = sem_d0 if p == 0 else sem_d1
                pend.append(pltpu.async_copy(
                    x8_v, out_hbm.at[b, j, pl.ds(c0, 8)], sem_c))
                if j >= 2:
                    pend[_idx_d[j - 2]].wait()

                @plsc.parallel_loop(0, _N, 16, unroll=8)
                def vbody(off):
                    vidx = idxt_v[j, pl.ds(off, 16)]
                    for r in range(8):
                        nb = plsc.load_gather(x8_v, [splats[r], vidx])
                        cv = x8_v[r, pl.ds(off, 16)]
                        dbuf_v[p, r, pl.ds(off, 16)] = cv - nb
                _idx_d[j] = len(pend)
                pend.append(pltpu.async_copy(
                    dbuf_v.at[p], out_hbm.at[b, j, pl.ds(_C + c0, 8)], sem_d))
            pend[_idx_d[_K - 2]].wait()
            pend[_idx_d[_K - 1]].wait()
            for j in range(_K):
                pend[2 * j].wait()
            return carry

        _idx_d = {}
        lax.fori_loop(0, _NOCT, oct_body, 0)

    return sc_gather


_sc_gather_cache = []


def kernel(x, k):
    del k
    if not _sc_gather_cache:
        _sc_gather_cache.append(_make_sc_gather())
    idx = _topk(x, x)
    idxt = jnp.transpose(idx, (0, 2, 1))
    phys = _sc_gather_cache[0](x, idxt)
    return jnp.transpose(phys, (0, 2, 3, 1))
